# baseline (speedup 1.0000x reference)
import functools

import jax
import jax.numpy as jnp
from jax import lax
from jax.experimental import pallas as pl
from jax.experimental.pallas import tpu as pltpu
from jax.experimental.pallas import tpu_sc as plsc

B, S, D = 1, 2048, 768
FF = 3072
E = 8
TT = 256
TTS = 256
NT = 24
NROWS = NT * TTS
NA = 2 * S
LANES = 16


def _gelu(v):
    return 0.5 * v * (1.0 + lax.erf(v * 0.7071067811865476))


def _router_shared_body(x_ref, wr_ref, w1_ref, b1_ref, w2_ref, b2_ref,
                        out_ref, idx2_ref, wtop_ref):
    xt = x_ref[:]
    logits = jnp.dot(xt.astype(jnp.bfloat16), wr_ref[:].astype(jnp.bfloat16),
                     preferred_element_type=jnp.float32)
    m = jnp.max(logits, axis=1, keepdims=True)
    p = jnp.exp(logits - m)
    p = p / jnp.sum(p, axis=1, keepdims=True)
    lane = lax.broadcasted_iota(jnp.int32, p.shape, 1)
    v1 = jnp.max(p, axis=1, keepdims=True)
    i1 = jnp.min(jnp.where(p == v1, lane, E), axis=1, keepdims=True)
    p2 = jnp.where(lane == i1, -1.0, p)
    v2 = jnp.max(p2, axis=1, keepdims=True)
    i2 = jnp.min(jnp.where(p2 == v2, lane, E), axis=1, keepdims=True)
    denom = v1 + v2
    idx2_ref[:] = jnp.concatenate([i1, i2], axis=1)
    wtop_ref[:] = jnp.concatenate([v1 / denom, v2 / denom], axis=1)

    xb = xt.astype(jnp.bfloat16)
    h = jnp.dot(xb, w1_ref[:].astype(jnp.bfloat16),
                preferred_element_type=jnp.float32) + b1_ref[:]
    h = _gelu(h)
    out_ref[:] = jnp.dot(h.astype(jnp.bfloat16), w2_ref[:].astype(jnp.bfloat16),
                         preferred_element_type=jnp.float32) + b2_ref[:]


def _cumsum16(v):
    lanes = lax.broadcasted_iota(jnp.int32, (LANES,), 0)
    for k in (1, 2, 4, 8):
        sh = v.at[jnp.maximum(lanes - k, 0)].get(mode="promise_in_bounds")
        v = v + jnp.where(lanes >= k, sh, 0)
    return v


def _splat(v, e):
    return v.at[jnp.full((LANES,), e, jnp.int32)].get(mode="promise_in_bounds")


def _dispatch_sort_body(idx2_hbm, wtop_hbm,
                        pos_a_hbm, pos_b_hbm, toks_hbm, wsort_hbm, expt_hbm,
                        idx_v, w_v, rel_v, pos_a_v, pos_b_v, toks_v, wsort_v,
                        expt_v):
    wid = lax.axis_index("s") * 2 + lax.axis_index("c")

    @pl.when(wid == 0)
    def _():
        pltpu.sync_copy(idx2_hbm, idx_v)
        pltpu.sync_copy(wtop_hbm, w_v)
        lanes = lax.broadcasted_iota(jnp.int32, (LANES,), 0)

        def rank_chunk(c, cur):
            ech = idx_v[pl.ds(c * LANES, LANES)]
            rel = jnp.zeros((LANES,), jnp.int32)
            add = jnp.zeros((LANES,), jnp.int32)
            for e in range(E):
                m = ech == e
                pref = _cumsum16(jnp.where(m, 1, 0))
                cnt = _splat(pref, LANES - 1)
                rel = jnp.where(m, _splat(cur, e) + pref - 1, rel)
                add = add + jnp.where(lanes == e, cnt, 0)
            rel_v[pl.ds(c * LANES, LANES)] = rel
            return cur + add

        counts = lax.fori_loop(0, NA // LANES, rank_chunk,
                               jnp.zeros((LANES,), jnp.int32))

        ntiles = (counts + (TTS - 1)) >> 8
        aligned = ntiles << 8
        base = _cumsum16(aligned) - aligned
        tbase = _cumsum16(ntiles) - ntiles

        for tc in range(NT // LANES + 1):
            tid = tc * LANES + lanes
            ex = jnp.zeros((LANES,), jnp.int32)
            for e in range(E):
                tb_e = _splat(tbase, e)
                nt_e = _splat(ntiles, e)
                ex = jnp.where((tid >= tb_e) & (tid < tb_e + nt_e), e, ex)
            expt_v[pl.ds(tc * LANES, LANES)] = ex

        def zero_chunk(c, _):
            toks_v[pl.ds(c * LANES, LANES)] = jnp.zeros((LANES,), jnp.int32)
            wsort_v[pl.ds(c * LANES, LANES)] = jnp.zeros((LANES,), jnp.float32)
            return 0
        lax.fori_loop(0, NROWS // LANES, zero_chunk, 0)

        def fin_chunk(c, _):
            ech = idx_v[pl.ds(c * LANES, LANES)]
            rel = rel_v[pl.ds(c * LANES, LANES)]
            pos = rel + base.at[ech].get(mode="promise_in_bounds")
            flat = c * LANES + lanes
            tok = flat >> 1
            even = (flat & 1) == 0
            plsc.store_scatter(toks_v, [pos], tok)
            plsc.store_scatter(wsort_v, [pos], w_v[pl.ds(c * LANES, LANES)])
            plsc.store_scatter(pos_a_v, [tok], pos, mask=even)
            plsc.store_scatter(pos_b_v, [tok], pos, mask=~even)
            return 0
        lax.fori_loop(0, NA // LANES, fin_chunk, 0)

        pltpu.sync_copy(pos_a_v, pos_a_hbm)
        pltpu.sync_copy(pos_b_v, pos_b_hbm)
        pltpu.sync_copy(toks_v, toks_hbm)
        pltpu.sync_copy(wsort_v, wsort_hbm)
        pltpu.sync_copy(expt_v, expt_hbm)


def _dispatch_gather_body(x_hbm, toks_hbm, xg_hbm, idx_v, rows_v, sem):
    wid = lax.axis_index("s") * 2 + lax.axis_index("c")
    for h in range(2):
        base = wid * (NROWS // 32) + h * (NROWS // 64)
        pltpu.sync_copy(toks_hbm.at[pl.ds(base, NROWS // 64)], idx_v)
        pltpu.async_copy(x_hbm.at[idx_v], rows_v, sem).wait()
        pltpu.sync_copy(rows_v, xg_hbm.at[pl.ds(base, NROWS // 64)])


def _group_ffn_body(expt_ref, xg_ref, w1_ref, b1_ref, w2_ref, b2_ref, wg_ref,
                    y_ref):
    xb = xg_ref[:].astype(jnp.bfloat16)
    h = jnp.dot(xb, w1_ref[0].astype(jnp.bfloat16),
                preferred_element_type=jnp.float32) + b1_ref[0]
    h = _gelu(h)
    o = jnp.dot(h.astype(jnp.bfloat16), w2_ref[0].astype(jnp.bfloat16),
                preferred_element_type=jnp.float32) + b2_ref[0]
    y_ref[:] = wg_ref[:] * o


def _combine_body(shared_hbm, yg_hbm, pos_a_hbm, pos_b_hbm, out_hbm,
                  pa_v, pb_v, ya_v, yb_v, acc_v, sem):
    wid = lax.axis_index("s") * 2 + lax.axis_index("c")
    for h in range(2):
        base = wid * (S // 32) + h * (S // 64)
        pltpu.sync_copy(pos_a_hbm.at[pl.ds(base, S // 64)], pa_v)
        pltpu.sync_copy(pos_b_hbm.at[pl.ds(base, S // 64)], pb_v)
        pltpu.sync_copy(shared_hbm.at[pl.ds(base, S // 64)], acc_v)
        cp_a = pltpu.async_copy(yg_hbm.at[pa_v], ya_v, sem)
        cp_b = pltpu.async_copy(yg_hbm.at[pb_v], yb_v, sem)
        cp_a.wait()
        cp_b.wait()

        def add_row(r, _):
            for j in range(D // LANES):
                sl = pl.ds(j * LANES, LANES)
                acc_v[r, sl] = acc_v[r, sl] + ya_v[r, sl] + yb_v[r, sl]
            return 0
        lax.fori_loop(0, S // 64, add_row, 0)
        pltpu.sync_copy(acc_v, out_hbm.at[pl.ds(base, S // 64)])


def kernel(x, Wr, sW1, sb1, sW2, sb2, pW1, pb1, pW2, pb2):
    xs = x.reshape(S, D)

    shared_out, idx2, wtop = pl.pallas_call(
        _router_shared_body,
        grid=(S // TT,),
        in_specs=[
            pl.BlockSpec((TT, D), lambda t: (t, 0)),
            pl.BlockSpec((D, E), lambda t: (0, 0)),
            pl.BlockSpec((D, FF), lambda t: (0, 0)),
            pl.BlockSpec((1, FF), lambda t: (0, 0)),
            pl.BlockSpec((FF, D), lambda t: (0, 0)),
            pl.BlockSpec((1, D), lambda t: (0, 0)),
        ],
        out_specs=[
            pl.BlockSpec((TT, D), lambda t: (t, 0)),
            pl.BlockSpec((TT, 2), lambda t: (t, 0)),
            pl.BlockSpec((TT, 2), lambda t: (t, 0)),
        ],
        out_shape=[
            jax.ShapeDtypeStruct((S, D), jnp.float32),
            jax.ShapeDtypeStruct((S, 2), jnp.int32),
            jax.ShapeDtypeStruct((S, 2), jnp.float32),
        ],
    )(xs, Wr, sW1, sb1.reshape(1, FF), sW2, sb2.reshape(1, D))

    mesh = plsc.VectorSubcoreMesh(core_axis_name="c", subcore_axis_name="s")

    dispatch = functools.partial(
        pl.kernel, mesh=mesh,
        compiler_params=pltpu.CompilerParams(needs_layout_passes=False),
        out_type=[
            jax.ShapeDtypeStruct((S,), jnp.int32),
            jax.ShapeDtypeStruct((S,), jnp.int32),
            jax.ShapeDtypeStruct((NROWS,), jnp.int32),
            jax.ShapeDtypeStruct((NROWS,), jnp.float32),
            jax.ShapeDtypeStruct((32,), jnp.int32),
        ],
        scratch_types=[
            pltpu.VMEM((NA,), jnp.int32),
            pltpu.VMEM((NA,), jnp.float32),
            pltpu.VMEM((NA,), jnp.int32),
            pltpu.VMEM((S,), jnp.int32),
            pltpu.VMEM((S,), jnp.int32),
            pltpu.VMEM((NROWS,), jnp.int32),
            pltpu.VMEM((NROWS,), jnp.float32),
            pltpu.VMEM((32,), jnp.int32),
        ],
    )(_dispatch_sort_body)
    pos_a, pos_b, toks, wsort, expt = dispatch(
        idx2.reshape(NA), wtop.reshape(NA))

    gather = functools.partial(
        pl.kernel, mesh=mesh,
        compiler_params=pltpu.CompilerParams(needs_layout_passes=False),
        out_type=jax.ShapeDtypeStruct((NROWS, D), jnp.float32),
        scratch_types=[
            pltpu.VMEM((NROWS // 64,), jnp.int32),
            pltpu.VMEM((NROWS // 64, D), jnp.float32),
            pltpu.SemaphoreType.DMA,
        ],
    )(_dispatch_gather_body)
    xg = gather(xs, toks)

    yg = pl.pallas_call(
        _group_ffn_body,
        grid_spec=pltpu.PrefetchScalarGridSpec(
            num_scalar_prefetch=1,
            grid=(NT,),
            in_specs=[
                pl.BlockSpec((TTS, D), lambda i, s: (i, 0)),
                pl.BlockSpec((1, D, FF), lambda i, s: (s[i], 0, 0)),
                pl.BlockSpec((1, 1, FF), lambda i, s: (s[i], 0, 0)),
                pl.BlockSpec((1, FF, D), lambda i, s: (s[i], 0, 0)),
                pl.BlockSpec((1, 1, D), lambda i, s: (s[i], 0, 0)),
                pl.BlockSpec((TTS, 1), lambda i, s: (i, 0)),
            ],
            out_specs=pl.BlockSpec((TTS, D), lambda i, s: (i, 0)),
        ),
        out_shape=jax.ShapeDtypeStruct((NROWS, D), jnp.float32),
    )(expt, xg, pW1, pb1.reshape(E, 1, FF), pW2, pb2.reshape(E, 1, D),
      wsort.reshape(NROWS, 1))

    combine = functools.partial(
        pl.kernel, mesh=mesh,
        compiler_params=pltpu.CompilerParams(needs_layout_passes=False),
        out_type=jax.ShapeDtypeStruct((S, D), jnp.float32),
        scratch_types=[
            pltpu.VMEM((S // 64,), jnp.int32),
            pltpu.VMEM((S // 64,), jnp.int32),
            pltpu.VMEM((S // 64, D), jnp.float32),
            pltpu.VMEM((S // 64, D), jnp.float32),
            pltpu.VMEM((S // 64, D), jnp.float32),
            pltpu.SemaphoreType.DMA,
        ],
    )(_combine_body)
    out = combine(shared_out, yg, pos_a, pos_b)

    return out.reshape(B, S, D)


# speedup vs baseline: 1.0013x; 1.0013x over previous
import functools

import jax
import jax.numpy as jnp
from jax import lax
from jax.experimental import pallas as pl
from jax.experimental.pallas import tpu as pltpu
from jax.experimental.pallas import tpu_sc as plsc

B, S, D = 1, 2048, 768
FF = 3072
E = 8
TT = 256
TTS = 256
NT = 24
NROWS = NT * TTS
NA = 2 * S
LANES = 16


def _gelu(v):
    return 0.5 * v * (1.0 + lax.erf(v * 0.7071067811865476))


def _router_shared_body(x_ref, wr_ref, w1_ref, b1_ref, w2_ref, b2_ref,
                        out_ref, idx2_ref, wtop_ref):
    xt = x_ref[:]
    logits = jnp.dot(xt.astype(jnp.bfloat16), wr_ref[:].astype(jnp.bfloat16),
                     preferred_element_type=jnp.float32)
    m = jnp.max(logits, axis=1, keepdims=True)
    p = jnp.exp(logits - m)
    p = p / jnp.sum(p, axis=1, keepdims=True)
    lane = lax.broadcasted_iota(jnp.int32, p.shape, 1)
    v1 = jnp.max(p, axis=1, keepdims=True)
    i1 = jnp.min(jnp.where(p == v1, lane, E), axis=1, keepdims=True)
    p2 = jnp.where(lane == i1, -1.0, p)
    v2 = jnp.max(p2, axis=1, keepdims=True)
    i2 = jnp.min(jnp.where(p2 == v2, lane, E), axis=1, keepdims=True)
    denom = v1 + v2
    idx2_ref[:] = jnp.concatenate([i1, i2], axis=1)
    wtop_ref[:] = jnp.concatenate([v1 / denom, v2 / denom], axis=1)

    xb = xt.astype(jnp.bfloat16)
    h = jnp.dot(xb, w1_ref[:].astype(jnp.bfloat16),
                preferred_element_type=jnp.float32) + b1_ref[:]
    h = _gelu(h)
    out_ref[:] = jnp.dot(h.astype(jnp.bfloat16), w2_ref[:].astype(jnp.bfloat16),
                         preferred_element_type=jnp.float32) + b2_ref[:]


def _cumsum16(v):
    lanes = lax.broadcasted_iota(jnp.int32, (LANES,), 0)
    for k in (1, 2, 4, 8):
        sh = v.at[jnp.maximum(lanes - k, 0)].get(mode="promise_in_bounds")
        v = v + jnp.where(lanes >= k, sh, 0)
    return v


def _splat(v, e):
    return v.at[jnp.full((LANES,), e, jnp.int32)].get(mode="promise_in_bounds")


def _dispatch_sort_body(idx2_hbm, wtop_hbm,
                        pos_a_hbm, pos_b_hbm, toks_hbm, wsort_hbm, expt_hbm,
                        idx_v, w_v, rel_v, pos_a_v, pos_b_v, toks_v, wsort_v,
                        expt_v):
    wid = lax.axis_index("s") * 2 + lax.axis_index("c")

    @pl.when(wid == 0)
    def _():
        pltpu.sync_copy(idx2_hbm, idx_v)
        pltpu.sync_copy(wtop_hbm, w_v)
        lanes = lax.broadcasted_iota(jnp.int32, (LANES,), 0)

        def rank_chunk(c, cur):
            ech = idx_v[pl.ds(c * LANES, LANES)]
            rel = jnp.zeros((LANES,), jnp.int32)
            add = jnp.zeros((LANES,), jnp.int32)
            for e in range(E):
                m = ech == e
                pref = _cumsum16(jnp.where(m, 1, 0))
                cnt = _splat(pref, LANES - 1)
                rel = jnp.where(m, _splat(cur, e) + pref - 1, rel)
                add = add + jnp.where(lanes == e, cnt, 0)
            rel_v[pl.ds(c * LANES, LANES)] = rel
            return cur + add

        counts = lax.fori_loop(0, NA // LANES, rank_chunk,
                               jnp.zeros((LANES,), jnp.int32))

        ntiles = (counts + (TTS - 1)) >> 8
        aligned = ntiles << 8
        base = _cumsum16(aligned) - aligned
        tbase = _cumsum16(ntiles) - ntiles

        for tc in range(NT // LANES + 1):
            tid = tc * LANES + lanes
            ex = jnp.zeros((LANES,), jnp.int32)
            for e in range(E):
                tb_e = _splat(tbase, e)
                nt_e = _splat(ntiles, e)
                ex = jnp.where((tid >= tb_e) & (tid < tb_e + nt_e), e, ex)
            expt_v[pl.ds(tc * LANES, LANES)] = ex

        def zero_chunk(c, _):
            toks_v[pl.ds(c * LANES, LANES)] = jnp.zeros((LANES,), jnp.int32)
            wsort_v[pl.ds(c * LANES, LANES)] = jnp.zeros((LANES,), jnp.float32)
            return 0
        lax.fori_loop(0, NROWS // LANES, zero_chunk, 0)

        def fin_chunk(c, _):
            ech = idx_v[pl.ds(c * LANES, LANES)]
            rel = rel_v[pl.ds(c * LANES, LANES)]
            pos = rel + base.at[ech].get(mode="promise_in_bounds")
            flat = c * LANES + lanes
            tok = flat >> 1
            even = (flat & 1) == 0
            plsc.store_scatter(toks_v, [pos], tok)
            plsc.store_scatter(wsort_v, [pos], w_v[pl.ds(c * LANES, LANES)])
            plsc.store_scatter(pos_a_v, [tok], pos, mask=even)
            plsc.store_scatter(pos_b_v, [tok], pos, mask=~even)
            return 0
        lax.fori_loop(0, NA // LANES, fin_chunk, 0)

        pltpu.sync_copy(pos_a_v, pos_a_hbm)
        pltpu.sync_copy(pos_b_v, pos_b_hbm)
        pltpu.sync_copy(toks_v, toks_hbm)
        pltpu.sync_copy(wsort_v, wsort_hbm)
        pltpu.sync_copy(expt_v, expt_hbm)


def _dispatch_gather_body(x_hbm, toks_hbm, xg_hbm, idx_v, rows_v, sem):
    wid = lax.axis_index("s") * 2 + lax.axis_index("c")
    for h in range(2):
        base = wid * (NROWS // 32) + h * (NROWS // 64)
        pltpu.sync_copy(toks_hbm.at[pl.ds(base, NROWS // 64)], idx_v)
        pltpu.async_copy(x_hbm.at[idx_v], rows_v, sem).wait()
        pltpu.sync_copy(rows_v, xg_hbm.at[pl.ds(base, NROWS // 64)])


def _group_ffn_body(expt_ref, xg_ref, w1_ref, b1_ref, w2_ref, b2_ref, wg_ref,
                    y_ref):
    xb = xg_ref[:].astype(jnp.bfloat16)
    h = jnp.dot(xb, w1_ref[0].astype(jnp.bfloat16),
                preferred_element_type=jnp.float32) + b1_ref[0]
    h = _gelu(h)
    o = jnp.dot(h.astype(jnp.bfloat16), w2_ref[0].astype(jnp.bfloat16),
                preferred_element_type=jnp.float32) + b2_ref[0]
    y_ref[:] = wg_ref[:] * o


def _combine_body(shared_hbm, yg_hbm, pos_a_hbm, pos_b_hbm, out_hbm,
                  pa_v, pb_v, ya_v, yb_v, acc_v, sem):
    wid = lax.axis_index("s") * 2 + lax.axis_index("c")
    for h in range(2):
        base = wid * (S // 32) + h * (S // 64)
        pltpu.sync_copy(pos_a_hbm.at[pl.ds(base, S // 64)], pa_v)
        pltpu.sync_copy(pos_b_hbm.at[pl.ds(base, S // 64)], pb_v)
        pltpu.sync_copy(shared_hbm.at[pl.ds(base, S // 64)], acc_v)
        cp_a = pltpu.async_copy(yg_hbm.at[pa_v], ya_v, sem)
        cp_b = pltpu.async_copy(yg_hbm.at[pb_v], yb_v, sem)
        cp_a.wait()
        cp_b.wait()

        def add_row(r, _):
            for j in range(D // LANES):
                sl = pl.ds(j * LANES, LANES)
                acc_v[r, sl] = acc_v[r, sl] + ya_v[r, sl] + yb_v[r, sl]
            return 0
        lax.fori_loop(0, S // 64, add_row, 0)
        pltpu.sync_copy(acc_v, out_hbm.at[pl.ds(base, S // 64)])


def kernel(x, Wr, sW1, sb1, sW2, sb2, pW1, pb1, pW2, pb2):
    xs = x.reshape(S, D)

    shared_out, idx2, wtop = pl.pallas_call(
        _router_shared_body,
        grid=(S // TT,),
        in_specs=[
            pl.BlockSpec((TT, D), lambda t: (t, 0)),
            pl.BlockSpec((D, E), lambda t: (0, 0)),
            pl.BlockSpec((D, FF), lambda t: (0, 0)),
            pl.BlockSpec((1, FF), lambda t: (0, 0)),
            pl.BlockSpec((FF, D), lambda t: (0, 0)),
            pl.BlockSpec((1, D), lambda t: (0, 0)),
        ],
        out_specs=[
            pl.BlockSpec((TT, D), lambda t: (t, 0)),
            pl.BlockSpec((TT, 2), lambda t: (t, 0)),
            pl.BlockSpec((TT, 2), lambda t: (t, 0)),
        ],
        out_shape=[
            jax.ShapeDtypeStruct((S, D), jnp.float32),
            jax.ShapeDtypeStruct((S, 2), jnp.int32),
            jax.ShapeDtypeStruct((S, 2), jnp.float32),
        ],
    )(xs, Wr, sW1, sb1.reshape(1, FF), sW2, sb2.reshape(1, D))

    mesh = plsc.VectorSubcoreMesh(core_axis_name="c", subcore_axis_name="s")

    dispatch = functools.partial(
        pl.kernel, mesh=mesh,
        compiler_params=pltpu.CompilerParams(needs_layout_passes=False),
        name="sc_dispatch_sort",
        out_type=[
            jax.ShapeDtypeStruct((S,), jnp.int32),
            jax.ShapeDtypeStruct((S,), jnp.int32),
            jax.ShapeDtypeStruct((NROWS,), jnp.int32),
            jax.ShapeDtypeStruct((NROWS,), jnp.float32),
            jax.ShapeDtypeStruct((32,), jnp.int32),
        ],
        scratch_types=[
            pltpu.VMEM((NA,), jnp.int32),
            pltpu.VMEM((NA,), jnp.float32),
            pltpu.VMEM((NA,), jnp.int32),
            pltpu.VMEM((S,), jnp.int32),
            pltpu.VMEM((S,), jnp.int32),
            pltpu.VMEM((NROWS,), jnp.int32),
            pltpu.VMEM((NROWS,), jnp.float32),
            pltpu.VMEM((32,), jnp.int32),
        ],
    )(_dispatch_sort_body)
    pos_a, pos_b, toks, wsort, expt = dispatch(
        idx2.reshape(NA), wtop.reshape(NA))

    gather = functools.partial(
        pl.kernel, mesh=mesh,
        compiler_params=pltpu.CompilerParams(needs_layout_passes=False),
        name="sc_dispatch_gather",
        out_type=jax.ShapeDtypeStruct((NROWS, D), jnp.float32),
        scratch_types=[
            pltpu.VMEM((NROWS // 64,), jnp.int32),
            pltpu.VMEM((NROWS // 64, D), jnp.float32),
            pltpu.SemaphoreType.DMA,
        ],
    )(_dispatch_gather_body)
    xg = gather(xs, toks)

    yg = pl.pallas_call(
        _group_ffn_body,
        grid_spec=pltpu.PrefetchScalarGridSpec(
            num_scalar_prefetch=1,
            grid=(NT,),
            in_specs=[
                pl.BlockSpec((TTS, D), lambda i, s: (i, 0)),
                pl.BlockSpec((1, D, FF), lambda i, s: (s[i], 0, 0)),
                pl.BlockSpec((1, 1, FF), lambda i, s: (s[i], 0, 0)),
                pl.BlockSpec((1, FF, D), lambda i, s: (s[i], 0, 0)),
                pl.BlockSpec((1, 1, D), lambda i, s: (s[i], 0, 0)),
                pl.BlockSpec((TTS, 1), lambda i, s: (i, 0)),
            ],
            out_specs=pl.BlockSpec((TTS, D), lambda i, s: (i, 0)),
        ),
        out_shape=jax.ShapeDtypeStruct((NROWS, D), jnp.float32),
    )(expt, xg, pW1, pb1.reshape(E, 1, FF), pW2, pb2.reshape(E, 1, D),
      wsort.reshape(NROWS, 1))

    combine = functools.partial(
        pl.kernel, mesh=mesh,
        compiler_params=pltpu.CompilerParams(needs_layout_passes=False),
        name="sc_combine",
        out_type=jax.ShapeDtypeStruct((S, D), jnp.float32),
        scratch_types=[
            pltpu.VMEM((S // 64,), jnp.int32),
            pltpu.VMEM((S // 64,), jnp.int32),
            pltpu.VMEM((S // 64, D), jnp.float32),
            pltpu.VMEM((S // 64, D), jnp.float32),
            pltpu.VMEM((S // 64, D), jnp.float32),
            pltpu.SemaphoreType.DMA,
        ],
    )(_combine_body)
    out = combine(shared_out, yg, pos_a, pos_b)

    return out.reshape(B, S, D)
